# Initial kernel scaffold; baseline (speedup 1.0000x reference)
import jax
import jax.numpy as jnp
from jax.experimental import pallas as pl


def kernel(x0, x1, x2, xbi0, xbi1, xbi2, W1, b1, W2, b2, Wb1, bb1, Wb2, bb2, table, weight_trans):
    raise NotImplementedError("write your pallas kernel here")


# speedup vs baseline: 2.0091x; 2.0091x over previous
import functools

import jax
import jax.numpy as jnp
from jax import lax
from jax.experimental import pallas as pl
from jax.experimental.pallas import tpu as pltpu
from jax.experimental.pallas import tpu_sc as plsc

NFEAT = 128
H = 64
NCLASS = 41
TDIM = 64
TNUM = 100000
B = 10000
N1 = 8
N2 = 2


_NC = 2
_NS = 16
_NW = _NC * _NS
_CH = 128
_NCHUNK = 20
_BPW = _CH * _NCHUNK
_BPAD = _BPW * _NW
_NBUF = 4


def _sc_gather_body(table_hbm, idx_hbm, out_hbm, idx_v, bufs0, bufs1, bufs2,
                    bufs3, gs0, gs1, gs2, gs3, os0, os1, os2, os3):
    bufs = [bufs0, bufs1, bufs2, bufs3]
    gsems = [gs0, gs1, gs2, gs3]
    osems = [os0, os1, os2, os3]
    wid = lax.axis_index("s") * _NC + lax.axis_index("c")
    base = wid * _BPW
    pltpu.sync_copy(idx_hbm.at[pl.ds(base, _BPW)], idx_v)
    gcp = [None] * _NBUF
    ocp = [None] * _NBUF
    for j in range(_NBUF):
        gcp[j] = pltpu.async_copy(
            table_hbm.at[idx_v.at[pl.ds(j * _CH, _CH)]], bufs[j], gsems[j])
    for i in range(_NCHUNK):
        b = i % _NBUF
        gcp[b].wait()
        ocp[b] = pltpu.async_copy(
            bufs[b], out_hbm.at[pl.ds(base + i * _CH, _CH)], osems[b])
        nxt = i + _NBUF
        if nxt < _NCHUNK:
            ocp[b].wait()
            gcp[b] = pltpu.async_copy(
                table_hbm.at[idx_v.at[pl.ds(nxt * _CH, _CH)]], bufs[b],
                gsems[b])
    for j in range(_NBUF):
        if _NCHUNK - _NBUF + j >= 0:
            ocp[(_NCHUNK - _NBUF + j) % _NBUF].wait()


def _sc_gather(table, idx_pad):
    mesh = plsc.VectorSubcoreMesh(core_axis_name="c", subcore_axis_name="s")
    f = pl.kernel(
        _sc_gather_body,
        mesh=mesh,
        compiler_params=pltpu.CompilerParams(use_tc_tiling_on_sc=False),
        out_type=jax.ShapeDtypeStruct((_BPAD, TDIM), jnp.float32),
        scratch_types=(
            [pltpu.VMEM((_BPW,), jnp.int32)]
            + [pltpu.VMEM((_CH, TDIM), jnp.float32) for _ in range(_NBUF)]
            + [pltpu.SemaphoreType.DMA for _ in range(2 * _NBUF)]
        ),
    )
    return f(table, idx_pad)


_P = 400
_GRID = B // _P


def _pre_body(x0r, x1r, x2r, xb0r, xb2r, W1r, b1r, W2r, b2r, Wtr,
              out1r, e0r, agge2r):
    prec = lax.Precision.HIGHEST
    W1 = W1r[...]
    b1 = b1r[...]
    Wt = Wtr[...]
    a1 = x1r[...].reshape(_P, N1, NFEAT).sum(axis=1) * (1.0 / N1)
    h0 = jnp.maximum(
        jnp.dot(x0r[...] + a1, W1, precision=prec) + b1, 0.0)
    a2 = x2r[...].reshape(_P * N1, N2, NFEAT).sum(axis=1) * (1.0 / N2)
    h1 = jnp.maximum(
        jnp.dot(x1r[...] + a2, W1, precision=prec) + b1, 0.0)
    ah1 = h1.reshape(_P, N1, H).sum(axis=1) * (1.0 / N1)
    out1r[...] = jnp.dot(h0 + ah1, W2r[...], precision=prec) + b2r[...]
    e0r[...] = jnp.maximum(jnp.dot(xb0r[...], Wt, precision=prec), 0.0)
    e2 = jnp.maximum(jnp.dot(xb2r[...], Wt, precision=prec), 0.0)
    agge2r[...] = e2.reshape(_P * N1, N2, TDIM).sum(axis=1) * (1.0 / N2)


def _dense_pre(x0, x1, x2, xbi0, xbi2, W1, b1, W2, b2, Wt):
    const = lambda shp: pl.BlockSpec(shp, lambda i: (0, 0))
    return pl.pallas_call(
        _pre_body,
        grid=(_GRID,),
        in_specs=[
            pl.BlockSpec((_P, NFEAT), lambda i: (i, 0)),
            pl.BlockSpec((_P * N1, NFEAT), lambda i: (i, 0)),
            pl.BlockSpec((_P * N1 * N2, NFEAT), lambda i: (i, 0)),
            pl.BlockSpec((_P, NFEAT), lambda i: (i, 0)),
            pl.BlockSpec((_P * N1 * N2, NFEAT), lambda i: (i, 0)),
            const((NFEAT, H)),
            const((1, H)),
            const((H, NCLASS)),
            const((1, NCLASS)),
            const((NFEAT, TDIM)),
        ],
        out_specs=[
            pl.BlockSpec((_P, NCLASS), lambda i: (i, 0)),
            pl.BlockSpec((_P, TDIM), lambda i: (i, 0)),
            pl.BlockSpec((_P * N1, TDIM), lambda i: (i, 0)),
        ],
        out_shape=[
            jax.ShapeDtypeStruct((B, NCLASS), jnp.float32),
            jax.ShapeDtypeStruct((B, TDIM), jnp.float32),
            jax.ShapeDtypeStruct((B * N1, TDIM), jnp.float32),
        ],
    )(x0, x1, x2, xbi0, xbi2, W1, b1.reshape(1, H), W2, b2.reshape(1, NCLASS),
      Wt)


def _final_body(e1r, e0r, agge2r, out1r, Wb1r, bb1r, Wb2r, bb2r, outr):
    prec = lax.Precision.HIGHEST
    Wb1 = Wb1r[...]
    bb1 = bb1r[...]
    e1 = e1r[...]
    ae1 = e1.reshape(_P, N1, TDIM).sum(axis=1) * (1.0 / N1)
    hb0 = jnp.maximum(
        jnp.dot(e0r[...] + ae1, Wb1, precision=prec) + bb1, 0.0)
    hb1 = jnp.maximum(
        jnp.dot(e1 + agge2r[...], Wb1, precision=prec) + bb1, 0.0)
    ahb1 = hb1.reshape(_P, N1, H).sum(axis=1) * (1.0 / N1)
    out2 = jnp.dot(hb0 + ahb1, Wb2r[...], precision=prec) + bb2r[...]
    o = 0.8 * out1r[...] + 0.2 * out2
    m = jnp.max(o, axis=1, keepdims=True)
    lse = jnp.log(jnp.sum(jnp.exp(o - m), axis=1, keepdims=True)) + m
    outr[...] = o - lse


def _final(e1, e0, agge2, out1, Wb1, bb1, Wb2, bb2):
    const = lambda shp: pl.BlockSpec(shp, lambda i: (0, 0))
    return pl.pallas_call(
        _final_body,
        grid=(_GRID,),
        in_specs=[
            pl.BlockSpec((_P * N1, TDIM), lambda i: (i, 0)),
            pl.BlockSpec((_P, TDIM), lambda i: (i, 0)),
            pl.BlockSpec((_P * N1, TDIM), lambda i: (i, 0)),
            pl.BlockSpec((_P, NCLASS), lambda i: (i, 0)),
            const((H, H)),
            const((1, H)),
            const((H, NCLASS)),
            const((1, NCLASS)),
        ],
        out_specs=pl.BlockSpec((_P, NCLASS), lambda i: (i, 0)),
        out_shape=jax.ShapeDtypeStruct((B, NCLASS), jnp.float32),
    )(e1, e0, agge2, out1, Wb1, bb1.reshape(1, H), Wb2, bb2.reshape(1, NCLASS))


def kernel(x0, x1, x2, xbi0, xbi1, xbi2, W1, b1, W2, b2, Wb1, bb1, Wb2, bb2,
           table, weight_trans):
    idx = xbi1.astype(jnp.int32)
    idx_pad = jnp.concatenate(
        [idx, jnp.zeros((_BPAD - B * N1,), jnp.int32)])
    e1 = _sc_gather(table, idx_pad)
    out1, e0, agge2 = _dense_pre(x0, x1, x2, xbi0, xbi2, W1, b1, W2, b2,
                                 weight_trans)
    return _final(e1, e0, agge2, out1, Wb1, bb1, Wb2, bb2)
